# baseline (speedup 1.0000x reference)
import jax
import jax.numpy as jnp
from jax import lax
from jax.experimental import pallas as pl
from jax.experimental.pallas import tpu as pltpu
from jax.experimental.pallas import tpu_sc as plsc

N = 10000
E = 320000
G = 64
D = 128
NCLS = 10

NC = 2
NS = 16
NT = NC * NS
CH = 128
NBUF = 2
NHALF = 2
EPT = E // NT
NCHUNK = 80
EPT_PAD = NCHUNK * CH
IDXH = NCHUNK // NHALF
NP = 10240
RPT = NP // NS
R = 1024

_mesh = plsc.VectorSubcoreMesh(
    core_axis_name="c", subcore_axis_name="s", num_cores=NC, num_subcores=NS)


def _deg_body(dst_hbm, out_hbm, dst_v, deg_v):
    cid = lax.axis_index("c")
    sid = lax.axis_index("s")
    wid = cid * NS + sid
    pltpu.sync_copy(dst_hbm.at[pl.ds(wid * EPT_PAD, EPT_PAD)], dst_v)

    def zero(i, _):
        deg_v[pl.ds(i * 16, 16)] = jnp.zeros((16,), jnp.float32)
        return 0
    lax.fori_loop(0, NP // 16, zero, 0)

    ones = jnp.ones((16,), jnp.float32)

    def body(i, _):
        idx = dst_v[pl.ds(i * 16, 16)]
        plsc.addupdate_scatter(deg_v, [idx], ones)
        return 0
    lax.fori_loop(0, EPT_PAD // 16, body, 0)
    pltpu.sync_copy(deg_v, out_hbm.at[wid])


_deg_call = pl.kernel(
    _deg_body,
    out_type=jax.ShapeDtypeStruct((NT, NP), jnp.float32),
    mesh=_mesh,
    scratch_types=[
        pltpu.VMEM((EPT_PAD,), jnp.int32),
        pltpu.VMEM((NP,), jnp.float32),
    ],
    compiler_params=pltpu.CompilerParams(needs_layout_passes=False),
)


def _scatter_body(hws_hbm, src_hbm, dst_hbm, out0_hbm, out1_hbm,
                  src_v, dst_v, rows_v, acc_sh, gsems, ssems):
    cid = lax.axis_index("c")
    sid = lax.axis_index("s")
    wid = cid * NS + sid

    def zrow(i, _):
        for j in range(D // 16):
            rows_v[0, i, pl.ds(j * 16, 16)] = jnp.zeros((16,), jnp.float32)
        return 0
    lax.fori_loop(0, CH, zrow, 0)
    row0 = sid * RPT
    for k in range(RPT // CH):
        pltpu.sync_copy(rows_v.at[0], acc_sh.at[pl.ds(row0 + k * CH, CH)])
    plsc.subcore_barrier()

    def body(g, _):
        base = g * NBUF
        gd = [pltpu.async_copy(hws_hbm.at[src_v.at[base + j]],
                               rows_v.at[j], gsems.at[j])
              for j in range(NBUF)]
        sd = []
        for j in range(NBUF):
            gd[j].wait()
            sd.append(pltpu.async_copy(rows_v.at[j],
                                       acc_sh.at[dst_v.at[base + j]],
                                       ssems.at[j], add=True))
        for j in range(NBUF):
            sd[j].wait()
        return 0

    for h in range(NHALF):
        pltpu.sync_copy(src_hbm.at[wid * NHALF + h], src_v)
        pltpu.sync_copy(dst_hbm.at[wid * NHALF + h], dst_v)
        lax.fori_loop(0, IDXH // NBUF, body, 0)
    plsc.subcore_barrier()

    @pl.when(cid == 0)
    def _():
        pltpu.sync_copy(acc_sh.at[pl.ds(row0, RPT)], out0_hbm.at[pl.ds(row0, RPT)])

    @pl.when(cid == 1)
    def _():
        pltpu.sync_copy(acc_sh.at[pl.ds(row0, RPT)], out1_hbm.at[pl.ds(row0, RPT)])


_scatter_call = pl.kernel(
    _scatter_body,
    out_type=[jax.ShapeDtypeStruct((NP, D), jnp.float32),
              jax.ShapeDtypeStruct((NP, D), jnp.float32)],
    mesh=_mesh,
    scratch_types=[
        pltpu.VMEM((IDXH, CH), jnp.int32),
        pltpu.VMEM((IDXH, CH), jnp.int32),
        pltpu.VMEM((NBUF, CH, D), jnp.float32),
        pltpu.VMEM_SHARED((NP, D), jnp.float32),
        pltpu.SemaphoreType.DMA((NBUF,)),
        pltpu.SemaphoreType.DMA((NBUF,)),
    ],
    compiler_params=pltpu.CompilerParams(needs_layout_passes=False),
)


NPS = 512


def _diag_body(hws_hbm, src_hbm, dst_hbm, out0_hbm, out1_hbm,
               src_v, dst_v, rows_v, stage_sh, acc_sh, gsems, ssems):
    cid = lax.axis_index("c")
    sid = lax.axis_index("s")
    wid = cid * NS + sid
    row0 = sid * RPT

    srow0 = sid * (NPS // NS)
    pltpu.sync_copy(hws_hbm.at[pl.ds(srow0, NPS // NS)],
                    rows_v.at[0, pl.ds(0, NPS // NS)])
    pltpu.sync_copy(rows_v.at[0, pl.ds(0, NPS // NS)],
                    stage_sh.at[pl.ds(srow0, NPS // NS)])

    def zrow(i, _):
        for j in range(D // 16):
            rows_v[0, i, pl.ds(j * 16, 16)] = jnp.zeros((16,), jnp.float32)
        return 0
    lax.fori_loop(0, CH, zrow, 0)
    for k in range(RPT // CH):
        pltpu.sync_copy(rows_v.at[0], acc_sh.at[pl.ds(row0 + k * CH, CH)])
    plsc.subcore_barrier()

    def body(g, _):
        base = g * NBUF
        gd = [pltpu.async_copy(stage_sh.at[src_v.at[base + j]],
                               rows_v.at[j], gsems.at[j])
              for j in range(NBUF)]
        sd = []
        for j in range(NBUF):
            gd[j].wait()
            sd.append(pltpu.async_copy(rows_v.at[j],
                                       acc_sh.at[dst_v.at[base + j]],
                                       ssems.at[j], add=True))
        for j in range(NBUF):
            sd[j].wait()
        return 0

    for h in range(NHALF):
        pltpu.sync_copy(src_hbm.at[wid * NHALF + h], src_v)
        pltpu.sync_copy(dst_hbm.at[wid * NHALF + h], dst_v)
        lax.fori_loop(0, IDXH // NBUF, body, 0)
    plsc.subcore_barrier()

    @pl.when(cid == 0)
    def _():
        pltpu.sync_copy(acc_sh.at[pl.ds(row0, RPT)], out0_hbm.at[pl.ds(row0, RPT)])

    @pl.when(cid == 1)
    def _():
        pltpu.sync_copy(acc_sh.at[pl.ds(row0, RPT)], out1_hbm.at[pl.ds(row0, RPT)])


_diag_call = pl.kernel(
    _diag_body,
    out_type=[jax.ShapeDtypeStruct((NP, D), jnp.float32),
              jax.ShapeDtypeStruct((NP, D), jnp.float32)],
    mesh=_mesh,
    scratch_types=[
        pltpu.VMEM((IDXH, CH), jnp.int32),
        pltpu.VMEM((IDXH, CH), jnp.int32),
        pltpu.VMEM((NBUF, CH, D), jnp.float32),
        pltpu.VMEM_SHARED((NPS, D), jnp.float32),
        pltpu.VMEM_SHARED((NP, D), jnp.float32),
        pltpu.SemaphoreType.DMA((NBUF,)),
        pltpu.SemaphoreType.DMA((NBUF,)),
    ],
    compiler_params=pltpu.CompilerParams(needs_layout_passes=False),
)


def _k1_body(h_ref, w_ref, p_ref, hws_ref):
    deg = jnp.sum(p_ref[...], axis=0) + 1.0
    dinv = lax.rsqrt(deg)[:, None]
    hw = jnp.dot(h_ref[...], w_ref[...], preferred_element_type=jnp.float32)
    hws_ref[...] = hw * dinv


_k1_call = pl.pallas_call(
    _k1_body,
    grid=(NP // R,),
    in_specs=[
        pl.BlockSpec((R, D), lambda i: (i, 0)),
        pl.BlockSpec((D, D), lambda i: (0, 0)),
        pl.BlockSpec((NT, R), lambda i: (0, i)),
    ],
    out_specs=pl.BlockSpec((R, D), lambda i: (i, 0)),
    out_shape=jax.ShapeDtypeStruct((NP, D), jnp.float32),
)


def _k2_body(a0_ref, a1_ref, hws_ref, p_ref, bias_ref, w_ref, out_ref):
    deg = jnp.sum(p_ref[...], axis=0) + 1.0
    dinv = lax.rsqrt(deg)[:, None]
    h = jnp.maximum(
        dinv * (a0_ref[...] + a1_ref[...] + hws_ref[...]) + bias_ref[...], 0.0)
    out_ref[...] = jnp.dot(
        h, w_ref[...], preferred_element_type=jnp.float32) * dinv


_k2_call = pl.pallas_call(
    _k2_body,
    grid=(NP // R,),
    in_specs=[
        pl.BlockSpec((R, D), lambda i: (i, 0)),
        pl.BlockSpec((R, D), lambda i: (i, 0)),
        pl.BlockSpec((R, D), lambda i: (i, 0)),
        pl.BlockSpec((NT, R), lambda i: (0, i)),
        pl.BlockSpec((1, D), lambda i: (0, 0)),
        pl.BlockSpec((D, D), lambda i: (0, 0)),
    ],
    out_specs=pl.BlockSpec((R, D), lambda i: (i, 0)),
    out_shape=jax.ShapeDtypeStruct((NP, D), jnp.float32),
)


def _k3_body(a0_ref, a1_ref, hws_ref, p_ref, bias_ref, bt_ref,
             wl_ref, bl_ref, wo_ref, bo_ref, out_ref, sums, counts):
    i = pl.program_id(0)

    @pl.when(i == 0)
    def _():
        sums[...] = jnp.zeros_like(sums)
        counts[...] = jnp.zeros_like(counts)

    deg = jnp.sum(p_ref[...], axis=0) + 1.0
    dinv = lax.rsqrt(deg)[:, None]
    h = jnp.maximum(
        dinv * (a0_ref[...] + a1_ref[...] + hws_ref[...]) + bias_ref[...], 0.0)
    giota = lax.broadcasted_iota(jnp.int32, (G, 128), 0)
    s_acc = sums[...]
    c_acc = counts[...]
    for rr in range(R // 128):
        bv = bt_ref[rr]
        oh = (giota == bv[None, :]).astype(jnp.float32)
        s_acc = s_acc + jnp.dot(oh, h[rr * 128:(rr + 1) * 128, :],
                                preferred_element_type=jnp.float32)
        c_acc = c_acc + jnp.sum(oh, axis=1, keepdims=True)
    sums[...] = s_acc
    counts[...] = c_acc

    @pl.when(i == pl.num_programs(0) - 1)
    def _():
        pooled = sums[...] / jnp.maximum(counts[...], 1.0)
        hl = jnp.maximum(
            jnp.dot(pooled, wl_ref[...], preferred_element_type=jnp.float32)
            + bl_ref[...], 0.0)
        out_ref[...] = jnp.dot(
            hl, wo_ref[...], preferred_element_type=jnp.float32) + bo_ref[...]


_k3_call = pl.pallas_call(
    _k3_body,
    grid=(NP // R,),
    in_specs=[
        pl.BlockSpec((R, D), lambda i: (i, 0)),
        pl.BlockSpec((R, D), lambda i: (i, 0)),
        pl.BlockSpec((R, D), lambda i: (i, 0)),
        pl.BlockSpec((NT, R), lambda i: (0, i)),
        pl.BlockSpec((1, D), lambda i: (0, 0)),
        pl.BlockSpec((R // 128, 128), lambda i: (i, 0)),
        pl.BlockSpec((D, D), lambda i: (0, 0)),
        pl.BlockSpec((1, D), lambda i: (0, 0)),
        pl.BlockSpec((D, D), lambda i: (0, 0)),
        pl.BlockSpec((1, D), lambda i: (0, 0)),
    ],
    out_specs=pl.BlockSpec((G, 128), lambda i: (0, 0)),
    out_shape=jax.ShapeDtypeStruct((G, 128), jnp.float32),
    scratch_shapes=[
        pltpu.VMEM((G, 128), jnp.float32),
        pltpu.VMEM((G, 128), jnp.float32),
    ],
)


def kernel(x, pos, edge_index, batch, W1, b1, W2, b2, Wl, bl, Wo, bo):
    h0 = jnp.concatenate([pos, x], axis=1)
    h0p = jnp.pad(h0, ((0, NP - N), (0, 0)))

    srcr = edge_index[0].reshape(NT, EPT)
    dstr = edge_index[1].reshape(NT, EPT)
    padc = jnp.full((NT, EPT_PAD - EPT), N, jnp.int32)
    src3 = jnp.concatenate([srcr, padc], axis=1).reshape(NT * NHALF, IDXH, CH)
    dst3 = jnp.concatenate([dstr, padc], axis=1).reshape(NT * NHALF, IDXH, CH)
    dst_flat = dst3.reshape(NT * EPT_PAD)

    batchp = jnp.pad(batch, (0, NP - N), constant_values=G).reshape(NP // 128, 128)
    b1r = b1.reshape(1, D)
    b2r = b2.reshape(1, D)
    wlp = jnp.pad(Wl, ((0, 0), (0, 128 - Wl.shape[1])))
    blr = jnp.pad(bl, (0, 128 - bl.shape[0])).reshape(1, 128)
    wop = jnp.pad(Wo, ((0, 128 - Wo.shape[0]), (0, 128 - Wo.shape[1])))
    bor = jnp.pad(bo, (0, 128 - bo.shape[0])).reshape(1, 128)

    partials = _deg_call(dst_flat)
    hws1 = _k1_call(h0p, W1, partials)
    acc1a, acc1b = _scatter_call(hws1, src3, dst3)
    hws2 = _k2_call(acc1a, acc1b, hws1, partials, b1r, W2)
    acc2a, acc2b = _scatter_call(hws2, src3, dst3)
    out64 = _k3_call(acc2a, acc2b, hws2, partials, b2r, batchp,
                     wlp, blr, wop, bor)

    srcs = jnp.bitwise_and(src3, NPS - 1)
    dsts = jnp.bitwise_and(dst3, NPS - 1)
    dga, dgb = _diag_call(hws1, srcs, dsts)
    out64 = out64 + 1e-30 * (dga[:G, :] + dgb[:G, :])
    return out64[:, :NCLS]


# speedup vs baseline: 1.1570x; 1.1570x over previous
import jax
import jax.numpy as jnp
from jax import lax
from jax.experimental import pallas as pl
from jax.experimental.pallas import tpu as pltpu
from jax.experimental.pallas import tpu_sc as plsc

N = 10000
E = 320000
G = 64
D = 128
NCLS = 10

NC = 2
NS = 16
NT = NC * NS
CH = 80
NBUF = 4
NHALF = 4
EPT = E // NT
NCHUNK = 128
EPT_PAD = NCHUNK * CH
IDXH = NCHUNK // NHALF
NP = 10240
RPT = NP // NS
R = 1024

_mesh = plsc.VectorSubcoreMesh(
    core_axis_name="c", subcore_axis_name="s", num_cores=NC, num_subcores=NS)


def _deg_body(dst_hbm, out_hbm, dst_v, deg_v):
    cid = lax.axis_index("c")
    sid = lax.axis_index("s")
    wid = cid * NS + sid
    pltpu.sync_copy(dst_hbm.at[pl.ds(wid * EPT_PAD, EPT_PAD)], dst_v)

    def zero(i, _):
        deg_v[pl.ds(i * 16, 16)] = jnp.zeros((16,), jnp.float32)
        return 0
    lax.fori_loop(0, NP // 16, zero, 0)

    ones = jnp.ones((16,), jnp.float32)

    def body(i, _):
        idx = dst_v[pl.ds(i * 16, 16)]
        plsc.addupdate_scatter(deg_v, [idx], ones)
        return 0
    lax.fori_loop(0, EPT_PAD // 16, body, 0)
    pltpu.sync_copy(deg_v, out_hbm.at[wid])


_deg_call = pl.kernel(
    _deg_body,
    out_type=jax.ShapeDtypeStruct((NT, NP), jnp.float32),
    mesh=_mesh,
    scratch_types=[
        pltpu.VMEM((EPT_PAD,), jnp.int32),
        pltpu.VMEM((NP,), jnp.float32),
    ],
    compiler_params=pltpu.CompilerParams(needs_layout_passes=False),
)


def _scatter_body(hws_hbm, src_hbm, dst_hbm, out0_hbm, out1_hbm,
                  src_v, dst_v, rows_v, acc_sh, gsems, ssems):
    cid = lax.axis_index("c")
    sid = lax.axis_index("s")
    wid = cid * NS + sid

    def zrow(i, _):
        for j in range(D // 16):
            rows_v[0, i, pl.ds(j * 16, 16)] = jnp.zeros((16,), jnp.float32)
        return 0
    lax.fori_loop(0, CH, zrow, 0)
    row0 = sid * RPT
    for k in range(RPT // CH):
        pltpu.sync_copy(rows_v.at[0], acc_sh.at[pl.ds(row0 + k * CH, CH)])
    plsc.subcore_barrier()

    def body(g, _):
        base = g * NBUF
        gd = [pltpu.async_copy(hws_hbm.at[src_v.at[base + j]],
                               rows_v.at[j], gsems.at[j])
              for j in range(NBUF)]
        sd = []
        for j in range(NBUF):
            gd[j].wait()
            sd.append(pltpu.async_copy(rows_v.at[j],
                                       acc_sh.at[dst_v.at[base + j]],
                                       ssems.at[j], add=True))
        for j in range(NBUF):
            sd[j].wait()
        return 0

    for h in range(NHALF):
        pltpu.sync_copy(src_hbm.at[wid * NHALF + h], src_v)
        pltpu.sync_copy(dst_hbm.at[wid * NHALF + h], dst_v)
        lax.fori_loop(0, IDXH // NBUF, body, 0)
    plsc.subcore_barrier()

    @pl.when(cid == 0)
    def _():
        pltpu.sync_copy(acc_sh.at[pl.ds(row0, RPT)], out0_hbm.at[pl.ds(row0, RPT)])

    @pl.when(cid == 1)
    def _():
        pltpu.sync_copy(acc_sh.at[pl.ds(row0, RPT)], out1_hbm.at[pl.ds(row0, RPT)])


_scatter_call = pl.kernel(
    _scatter_body,
    out_type=[jax.ShapeDtypeStruct((NP, D), jnp.float32),
              jax.ShapeDtypeStruct((NP, D), jnp.float32)],
    mesh=_mesh,
    scratch_types=[
        pltpu.VMEM((IDXH, CH), jnp.int32),
        pltpu.VMEM((IDXH, CH), jnp.int32),
        pltpu.VMEM((NBUF, CH, D), jnp.float32),
        pltpu.VMEM_SHARED((NP, D), jnp.float32),
        pltpu.SemaphoreType.DMA((NBUF,)),
        pltpu.SemaphoreType.DMA((NBUF,)),
    ],
    compiler_params=pltpu.CompilerParams(needs_layout_passes=False),
)


def _k1_body(h_ref, w_ref, p_ref, hws_ref):
    deg = jnp.sum(p_ref[...], axis=0) + 1.0
    dinv = lax.rsqrt(deg)[:, None]
    hw = jnp.dot(h_ref[...], w_ref[...], preferred_element_type=jnp.float32)
    hws_ref[...] = hw * dinv


_k1_call = pl.pallas_call(
    _k1_body,
    grid=(NP // R,),
    in_specs=[
        pl.BlockSpec((R, D), lambda i: (i, 0)),
        pl.BlockSpec((D, D), lambda i: (0, 0)),
        pl.BlockSpec((NT, R), lambda i: (0, i)),
    ],
    out_specs=pl.BlockSpec((R, D), lambda i: (i, 0)),
    out_shape=jax.ShapeDtypeStruct((NP, D), jnp.float32),
)


def _k2_body(a0_ref, a1_ref, hws_ref, p_ref, bias_ref, w_ref, out_ref):
    deg = jnp.sum(p_ref[...], axis=0) + 1.0
    dinv = lax.rsqrt(deg)[:, None]
    h = jnp.maximum(
        dinv * (a0_ref[...] + a1_ref[...] + hws_ref[...]) + bias_ref[...], 0.0)
    out_ref[...] = jnp.dot(
        h, w_ref[...], preferred_element_type=jnp.float32) * dinv


_k2_call = pl.pallas_call(
    _k2_body,
    grid=(NP // R,),
    in_specs=[
        pl.BlockSpec((R, D), lambda i: (i, 0)),
        pl.BlockSpec((R, D), lambda i: (i, 0)),
        pl.BlockSpec((R, D), lambda i: (i, 0)),
        pl.BlockSpec((NT, R), lambda i: (0, i)),
        pl.BlockSpec((1, D), lambda i: (0, 0)),
        pl.BlockSpec((D, D), lambda i: (0, 0)),
    ],
    out_specs=pl.BlockSpec((R, D), lambda i: (i, 0)),
    out_shape=jax.ShapeDtypeStruct((NP, D), jnp.float32),
)


def _k3_body(a0_ref, a1_ref, hws_ref, p_ref, bias_ref, bt_ref,
             wl_ref, bl_ref, wo_ref, bo_ref, out_ref, sums, counts):
    i = pl.program_id(0)

    @pl.when(i == 0)
    def _():
        sums[...] = jnp.zeros_like(sums)
        counts[...] = jnp.zeros_like(counts)

    deg = jnp.sum(p_ref[...], axis=0) + 1.0
    dinv = lax.rsqrt(deg)[:, None]
    h = jnp.maximum(
        dinv * (a0_ref[...] + a1_ref[...] + hws_ref[...]) + bias_ref[...], 0.0)
    giota = lax.broadcasted_iota(jnp.int32, (G, 128), 0)
    s_acc = sums[...]
    c_acc = counts[...]
    for rr in range(R // 128):
        bv = bt_ref[rr]
        oh = (giota == bv[None, :]).astype(jnp.float32)
        s_acc = s_acc + jnp.dot(oh, h[rr * 128:(rr + 1) * 128, :],
                                preferred_element_type=jnp.float32)
        c_acc = c_acc + jnp.sum(oh, axis=1, keepdims=True)
    sums[...] = s_acc
    counts[...] = c_acc

    @pl.when(i == pl.num_programs(0) - 1)
    def _():
        pooled = sums[...] / jnp.maximum(counts[...], 1.0)
        hl = jnp.maximum(
            jnp.dot(pooled, wl_ref[...], preferred_element_type=jnp.float32)
            + bl_ref[...], 0.0)
        out_ref[...] = jnp.dot(
            hl, wo_ref[...], preferred_element_type=jnp.float32) + bo_ref[...]


_k3_call = pl.pallas_call(
    _k3_body,
    grid=(NP // R,),
    in_specs=[
        pl.BlockSpec((R, D), lambda i: (i, 0)),
        pl.BlockSpec((R, D), lambda i: (i, 0)),
        pl.BlockSpec((R, D), lambda i: (i, 0)),
        pl.BlockSpec((NT, R), lambda i: (0, i)),
        pl.BlockSpec((1, D), lambda i: (0, 0)),
        pl.BlockSpec((R // 128, 128), lambda i: (i, 0)),
        pl.BlockSpec((D, D), lambda i: (0, 0)),
        pl.BlockSpec((1, D), lambda i: (0, 0)),
        pl.BlockSpec((D, D), lambda i: (0, 0)),
        pl.BlockSpec((1, D), lambda i: (0, 0)),
    ],
    out_specs=pl.BlockSpec((G, 128), lambda i: (0, 0)),
    out_shape=jax.ShapeDtypeStruct((G, 128), jnp.float32),
    scratch_shapes=[
        pltpu.VMEM((G, 128), jnp.float32),
        pltpu.VMEM((G, 128), jnp.float32),
    ],
)


def kernel(x, pos, edge_index, batch, W1, b1, W2, b2, Wl, bl, Wo, bo):
    h0 = jnp.concatenate([pos, x], axis=1)
    h0p = jnp.pad(h0, ((0, NP - N), (0, 0)))

    srcr = edge_index[0].reshape(NT, EPT)
    dstr = edge_index[1].reshape(NT, EPT)
    padc = jnp.full((NT, EPT_PAD - EPT), N, jnp.int32)
    src3 = jnp.concatenate([srcr, padc], axis=1).reshape(NT * NHALF, IDXH, CH)
    dst3 = jnp.concatenate([dstr, padc], axis=1).reshape(NT * NHALF, IDXH, CH)
    dst_flat = dst3.reshape(NT * EPT_PAD)

    batchp = jnp.pad(batch, (0, NP - N), constant_values=G).reshape(NP // 128, 128)
    b1r = b1.reshape(1, D)
    b2r = b2.reshape(1, D)
    wlp = jnp.pad(Wl, ((0, 0), (0, 128 - Wl.shape[1])))
    blr = jnp.pad(bl, (0, 128 - bl.shape[0])).reshape(1, 128)
    wop = jnp.pad(Wo, ((0, 128 - Wo.shape[0]), (0, 128 - Wo.shape[1])))
    bor = jnp.pad(bo, (0, 128 - bo.shape[0])).reshape(1, 128)

    partials = _deg_call(dst_flat)
    hws1 = _k1_call(h0p, W1, partials)
    acc1a, acc1b = _scatter_call(hws1, src3, dst3)
    hws2 = _k2_call(acc1a, acc1b, hws1, partials, b1r, W2)
    acc2a, acc2b = _scatter_call(hws2, src3, dst3)
    out64 = _k3_call(acc2a, acc2b, hws2, partials, b2r, batchp,
                     wlp, blr, wop, bor)
    return out64[:, :NCLS]
